# Initial kernel scaffold; baseline (speedup 1.0000x reference)
import jax
import jax.numpy as jnp
from jax.experimental import pallas as pl


def kernel(user_indices, query_indices, item_indices, queries_bag, queries_bag_offset, W_user, W_item, W_vocab):
    raise NotImplementedError("write your pallas kernel here")


# speedup vs baseline: 18.9397x; 18.9397x over previous
import functools
import jax
import jax.numpy as jnp
from jax import lax
from jax.experimental import pallas as pl
from jax.experimental.pallas import tpu as pltpu
from jax.experimental.pallas import tpu_sc as plsc

D = 64
LANES = 16
NC = 2
NS = 16
NW = NC * NS
GW = 128


def _body(uidx_hbm, qidx_hbm, iidx_hbm, bag_hbm, offs_hbm,
          wu_hbm, wi_hbm, wv_hbm,
          out_u, out_q, out_i,
          qbuf, sbuf, ebuf, idx2d, rowsbuf, qacc,
          bagbuf, widx, wrows, w0row):
    B = uidx_hbm.shape[0]
    NQ = offs_hbm.shape[0]
    L = bag_hbm.shape[0]
    BPW = B // NW
    NCHK = BPW // LANES
    NG = BPW // GW
    CPG = GW // LANES

    wid = lax.axis_index("s") * NC + lax.axis_index("c")
    base = wid * BPW
    lane = lax.iota(jnp.int32, LANES)

    def stage_idx(c, v):
        idx2d[c // CPG, pl.ds((c % CPG) * LANES, LANES)] = v

    for src, tbl, dst in ((uidx_hbm, wu_hbm, out_u), (iidx_hbm, wi_hbm, out_i)):
        pltpu.sync_copy(src.at[pl.ds(base, BPW)], qbuf)
        for c in range(NCHK):
            stage_idx(c, qbuf[pl.ds(c * LANES, LANES)] + 1)
        for k in range(NG):
            pltpu.sync_copy(tbl.at[idx2d.at[k]],
                            rowsbuf.at[pl.ds(k * GW, GW)])
        pltpu.sync_copy(rowsbuf, dst.at[pl.ds(base, BPW)])

    pltpu.sync_copy(qidx_hbm.at[pl.ds(base, BPW)], qbuf)
    for c in range(NCHK):
        stage_idx(c, qbuf[pl.ds(c * LANES, LANES)])
    for k in range(NG):
        pltpu.sync_copy(offs_hbm.at[idx2d.at[k]], sbuf.at[pl.ds(k * GW, GW)])
    for c in range(NCHK):
        stage_idx(c, jnp.minimum(qbuf[pl.ds(c * LANES, LANES)] + 1, NQ - 1))
    for k in range(NG):
        pltpu.sync_copy(offs_hbm.at[idx2d.at[k]], ebuf.at[pl.ds(k * GW, GW)])
    for c in range(NCHK):
        qv = qbuf[pl.ds(c * LANES, LANES)]
        ev = ebuf[pl.ds(c * LANES, LANES)]
        ebuf[pl.ds(c * LANES, LANES)] = jnp.where(qv == NQ - 1, L, ev)

    pltpu.sync_copy(wv_hbm.at[pl.ds(0, 1)], w0row)

    def getscalar(ref, b):
        return ref[pl.ds(b, LANES)][0]

    def per_query(b, _):
        s = getscalar(sbuf, b)
        e = getscalar(ebuf, b)
        cnt = e - s
        nch = (cnt + (LANES - 1)) // LANES

        def chunk(c, acc):
            pos = s + c * LANES
            abase = jnp.minimum((pos // 8) * 8, L - 40)
            off = pos - abase
            pltpu.sync_copy(bag_hbm.at[pl.ds(abase, 40)],
                            bagbuf.at[pl.ds(0, 40)])
            vals = bagbuf[pl.ds(off, LANES)]
            nvalid = jnp.minimum(cnt - c * LANES, LANES)
            widx[...] = jnp.where(lane < nvalid, vals, 0)
            pltpu.sync_copy(wv_hbm.at[widx], wrows)
            for r in range(LANES):
                acc = tuple(acc[j] + wrows[r, pl.ds(j * LANES, LANES)]
                            for j in range(D // LANES))
            return acc

        zero = jnp.zeros((LANES,), jnp.float32)
        acc = lax.fori_loop(0, nch, chunk, (zero,) * (D // LANES))

        ninvf = (nch * LANES - cnt).astype(jnp.float32)
        cnt_v = jnp.full((LANES,), 0, jnp.int32) + jnp.maximum(cnt, 1)
        scale = 1.0 / cnt_v.astype(jnp.float32)
        for j in range(D // LANES):
            mean = (acc[j] - ninvf * w0row[0, pl.ds(j * LANES, LANES)]) * scale
            qacc[b, pl.ds(j * LANES, LANES)] = mean
        return 0

    lax.fori_loop(0, BPW, per_query, 0)
    pltpu.sync_copy(qacc, out_q.at[pl.ds(base, BPW)])


@jax.jit
def kernel(user_indices, query_indices, item_indices, queries_bag,
           queries_bag_offset, W_user, W_item, W_vocab):
    B = user_indices.shape[0]
    BPW = B // NW
    mesh = plsc.VectorSubcoreMesh(core_axis_name="c", subcore_axis_name="s",
                                  num_cores=NC, num_subcores=NS)
    out_t = jax.ShapeDtypeStruct((B, D), jnp.float32)
    fn = pl.kernel(
        _body,
        out_type=(out_t, out_t, out_t),
        mesh=mesh,
        compiler_params=pltpu.CompilerParams(use_tc_tiling_on_sc=False),
        scratch_types=[
            pltpu.VMEM((BPW,), jnp.int32),
            pltpu.VMEM((BPW + LANES,), jnp.int32),
            pltpu.VMEM((BPW + LANES,), jnp.int32),
            pltpu.VMEM((BPW // GW, GW), jnp.int32),
            pltpu.VMEM((BPW, D), jnp.float32),
            pltpu.VMEM((BPW, D), jnp.float32),
            pltpu.VMEM((56,), jnp.int32),
            pltpu.VMEM((LANES,), jnp.int32),
            pltpu.VMEM((LANES, D), jnp.float32),
            pltpu.VMEM((1, D), jnp.float32),
        ],
    )
    return fn(user_indices.astype(jnp.int32),
              query_indices.astype(jnp.int32),
              item_indices.astype(jnp.int32),
              queries_bag.astype(jnp.int32),
              queries_bag_offset.astype(jnp.int32),
              W_user, W_item, W_vocab)
